# baseline (speedup 1.0000x reference)
import functools

import jax
import jax.numpy as jnp
from jax import lax
from jax.experimental import pallas as pl
from jax.experimental.pallas import tpu as pltpu
from jax.experimental.pallas import tpu_sc as plsc

NC = 2
NS = 16
NW = NC * NS
L = 16

BB = 128
PITCH = BB + 1
GDEPTH = 4


def _repack_body(V, D, tt_hbm, tail_hbm, out_hbm, src_v, dst_v,
                 rsem0, rsem1, osem0, osem1):
  wid = lax.axis_index("s") * NC + lax.axis_index("c")
  nblk = V // BB
  per_w_lo = nblk // NW
  n_extra = nblk - per_w_lo * NW
  base = wid * per_w_lo + jnp.minimum(wid, n_extra)
  n = per_w_lo + jnp.where(wid < n_extra, 1, 0)
  nslots = per_w_lo + (1 if n_extra else 0)
  nslots += nslots % 2
  lines_per_blk = BB // 4

  iota = lax.broadcasted_iota(jnp.int32, (L,), 0)
  idx_d = [iota + L * k for k in range(D // L)]
  rsems = (rsem0, rsem1)
  osems = (osem0, osem1)

  def fire_read(s, p):
    pltpu.async_copy(tt_hbm.at[:, pl.ds((base + s) * BB, BB)],
                     src_v.at[p, :, pl.ds(0, BB)], rsems[p])

  def wait_read(s, p):
    pltpu.make_async_copy(tt_hbm.at[:, pl.ds((base + s) * BB, BB)],
                          src_v.at[p, :, pl.ds(0, BB)], rsems[p]).wait()

  def transpose_cols(p, ncols):
    src = src_v.at[p]
    dst = dst_v.at[p]

    def _col(v, carry):
      vvec = jnp.full((L,), v, dtype=jnp.int32)
      for k in range(D // L):
        val = plsc.load_gather(src, [idx_d[k], vvec])
        dst[v // 4, pl.ds((v % 4) * D + L * k, L)] = val
      return carry

    lax.fori_loop(0, ncols, _col, 0, unroll=8)

  def fire_wb(s, p):
    pltpu.async_copy(dst_v.at[p],
                     out_hbm.at[pl.ds((base + s) * lines_per_blk,
                                      lines_per_blk)], osems[p])

  def wait_wb(s, p):
    pltpu.make_async_copy(dst_v.at[p],
                          out_hbm.at[pl.ds((base + s) * lines_per_blk,
                                           lines_per_blk)], osems[p]).wait()

  @pl.when(n > 0)
  def _prologue():
    fire_read(0, 0)

  def pair_body(i0, carry):
    for b in range(2):
      s = i0 + b
      @pl.when(s < n)
      def _slot():
        @pl.when(s + 1 < n)
        def _():
          fire_read(s + 1, (b + 1) % 2)
        wait_read(s, b)
        @pl.when(s >= 2)
        def _():
          wait_wb(s, b)
        transpose_cols(b, BB)
        fire_wb(s, b)
    return carry

  lax.fori_loop(0, nslots // 2, lambda i, c: pair_body(i * 2, c), 0,
                unroll=False)
  for b in range(2):
    @pl.when(n > b)
    def _drain():
      wait_wb(0, b)

  tail = V - nblk * BB
  if tail:
    @pl.when(wid == NW - 1)
    def _tail():
      pltpu.sync_copy(tail_hbm, dst_v.at[0, pl.ds(0, tail // 4), :])
      pltpu.sync_copy(dst_v.at[0, pl.ds(0, tail // 4), :],
                      out_hbm.at[pl.ds(nblk * lines_per_blk, tail // 4)])


def _emb_body(H, D, x_hbm, table_hbm, out_hbm, idx_v, rows_v, t_v,
              gsem0, gsem1, gsem2, gsem3, wsem0, wsem1):
  wid = lax.axis_index("s") * NC + lax.axis_index("c")
  pltpu.sync_copy(x_hbm.at[wid], idx_v)

  iota = lax.broadcasted_iota(jnp.int32, (L,), 0)
  idx_d = [iota + L * k for k in range(D // L)]
  gsems = (gsem0, gsem1, gsem2, gsem3)
  wsems = (wsem0, wsem1)
  nt = D // 8

  def fire_gather(h, g):
    pltpu.async_copy(table_hbm.at[idx_v.at[h]], rows_v.at[g], gsems[g])

  def wait_gather(h, g):
    pltpu.make_async_copy(table_hbm.at[idx_v.at[h]], rows_v.at[g],
                          gsems[g]).wait()

  def transpose_block(g, p):
    src = rows_v.at[g]
    dst = t_v.at[p]

    def _row(r, carry):
      rvec = jnp.full((L,), r, dtype=jnp.int32)
      for k in range(D // L):
        val = src[r, pl.ds(L * k, L)]
        plsc.store_scatter(dst, [idx_d[k], rvec], val)
      return carry

    lax.fori_loop(0, BB, _row, 0, unroll=8)

  def fire_wb(h, p):
    for ti in range(nt):
      pltpu.async_copy(t_v.at[p, pl.ds(ti * 8, 8), pl.ds(0, BB)],
                       out_hbm.at[h, ti, wid], wsems[p])

  def wait_wb(h, p):
    for ti in range(nt):
      pltpu.make_async_copy(t_v.at[p, pl.ds(ti * 8, 8), pl.ds(0, BB)],
                            out_hbm.at[h, ti, wid], wsems[p]).wait()

  for h in range(GDEPTH):
    fire_gather(h, h)

  def quad_body(h0, carry):
    for b in range(GDEPTH):
      h = h0 + b
      p = b % 2
      wait_gather(h, b)
      @pl.when(h >= 2)
      def _drain_prev_wb():
        wait_wb(h, p)
      transpose_block(b, p)
      @pl.when(h + GDEPTH < H)
      def _fire_next():
        fire_gather(h + GDEPTH, b)
      fire_wb(h, p)
    return carry

  lax.fori_loop(0, H // GDEPTH, lambda i, c: quad_body(i * GDEPTH, c), 0,
                unroll=False)

  for p in range(2):
    wait_wb(0, p)


def kernel(x, table):
  B, H = x.shape
  V, D = table.shape
  assert B == NW * BB and D % 8 == 0 and H % GDEPTH == 0
  nt = D // 8

  mesh = plsc.VectorSubcoreMesh(core_axis_name="c", subcore_axis_name="s")

  repack = pl.kernel(
      functools.partial(_repack_body, V, D),
      out_type=jax.ShapeDtypeStruct((V // 4, 4 * D), jnp.float32),
      mesh=mesh,
      scratch_types=[
          pltpu.VMEM((2, D, PITCH), jnp.float32),
          pltpu.VMEM((2, BB // 4, 4 * D), jnp.float32),
          pltpu.SemaphoreType.DMA,
          pltpu.SemaphoreType.DMA,
          pltpu.SemaphoreType.DMA,
          pltpu.SemaphoreType.DMA,
      ],
      compiler_params=pltpu.CompilerParams(use_tc_tiling_on_sc=True,
                                           needs_layout_passes=False),
  )

  xq = x.T.reshape(H, NW, BB).transpose(1, 0, 2)
  tail = V % BB
  t_tail = table[V - tail:].reshape(tail // 4, 4 * D)
  t32 = repack(table.T, t_tail).reshape(V, D)

  grid_kernel = pl.kernel(
      functools.partial(_emb_body, H, D),
      out_type=jax.ShapeDtypeStruct((H, nt, NW, 8, BB), jnp.float32),
      mesh=mesh,
      scratch_types=[
          pltpu.VMEM((H, BB), jnp.int32),
          pltpu.VMEM((GDEPTH, BB, D), jnp.float32),
          pltpu.VMEM((2, D, PITCH), jnp.float32),
          pltpu.SemaphoreType.DMA,
          pltpu.SemaphoreType.DMA,
          pltpu.SemaphoreType.DMA,
          pltpu.SemaphoreType.DMA,
          pltpu.SemaphoreType.DMA,
          pltpu.SemaphoreType.DMA,
      ],
      compiler_params=pltpu.CompilerParams(use_tc_tiling_on_sc=False,
                                           needs_layout_passes=False),
  )
  out5 = grid_kernel(xq, t32)
  return out5.transpose(2, 4, 0, 1, 3).reshape(B, H, D)


# speedup vs baseline: 1.3530x; 1.3530x over previous
import functools

import jax
import jax.numpy as jnp
from jax import lax
from jax.experimental import pallas as pl
from jax.experimental.pallas import tpu as pltpu
from jax.experimental.pallas import tpu_sc as plsc

NC = 2
NS = 16
NW = NC * NS
L = 16

BB = 128
PITCH = BB + 1
GDEPTH = 4


def _split_work(wid, total):
  per_w_lo = total // NW
  n_extra = total - per_w_lo * NW
  base = wid * per_w_lo + jnp.minimum(wid, n_extra)
  n = per_w_lo + jnp.where(wid < n_extra, 1, 0)
  nslots = per_w_lo + (1 if n_extra else 0)
  nslots += nslots % 2
  return base, n, nslots


def _relocate_body(V, D, tt_hbm, out_hbm, buf_v, rsem0, rsem1,
                   osem0, osem1):
  wid = lax.axis_index("s") * NC + lax.axis_index("c")
  nblk = V // BB
  base, n, nslots = _split_work(wid, nblk)
  rsems = (rsem0, rsem1)
  osems = (osem0, osem1)

  def fire_read(s, p):
    pltpu.async_copy(tt_hbm.at[:, pl.ds((base + s) * BB, BB)],
                     buf_v.at[p], rsems[p])

  def wait_read(s, p):
    pltpu.make_async_copy(tt_hbm.at[:, pl.ds((base + s) * BB, BB)],
                          buf_v.at[p], rsems[p]).wait()

  def fire_wb(s, p):
    pltpu.async_copy(buf_v.at[p], out_hbm.at[base + s], osems[p])

  def wait_wb(s, p):
    pltpu.make_async_copy(buf_v.at[p], out_hbm.at[base + s],
                          osems[p]).wait()

  @pl.when(n > 0)
  def _prologue():
    fire_read(0, 0)

  def pair_body(i0, carry):
    for b in range(2):
      s = i0 + b
      @pl.when(s < n)
      def _slot():
        @pl.when(s + 1 < n)
        def _():
          fire_read(s + 1, (b + 1) % 2)
        wait_read(s, b)
        @pl.when(s >= 2)
        def _():
          wait_wb(s, b)
        fire_wb(s, b)
    return carry

  lax.fori_loop(0, nslots // 2, lambda i, c: pair_body(i * 2, c), 0,
                unroll=False)
  for b in range(2):
    @pl.when(n > b)
    def _drain():
      wait_wb(0, b)


def _transpose_body(V, D, raw_hbm, tail_hbm, out_hbm, src_v, dst_v,
                    rsem0, rsem1, osem0, osem1):
  wid = lax.axis_index("s") * NC + lax.axis_index("c")
  nblk = V // BB
  base, n, nslots = _split_work(wid, nblk)
  lines_per_blk = BB // 4

  iota = lax.broadcasted_iota(jnp.int32, (L,), 0)
  idx_d = [iota + L * k for k in range(D // L)]
  rsems = (rsem0, rsem1)
  osems = (osem0, osem1)

  def fire_read(s, p):
    pltpu.async_copy(raw_hbm.at[base + s], src_v.at[p, :, pl.ds(0, BB)],
                     rsems[p])

  def wait_read(s, p):
    pltpu.make_async_copy(raw_hbm.at[base + s],
                          src_v.at[p, :, pl.ds(0, BB)], rsems[p]).wait()

  def transpose_cols(p):
    src = src_v.at[p]
    dst = dst_v.at[p]

    def _col(v, carry):
      vvec = jnp.full((L,), v, dtype=jnp.int32)
      for k in range(D // L):
        val = plsc.load_gather(src, [idx_d[k], vvec])
        dst[v // 4, pl.ds((v % 4) * D + L * k, L)] = val
      return carry

    lax.fori_loop(0, BB, _col, 0, unroll=8)

  def fire_wb(s, p):
    pltpu.async_copy(dst_v.at[p],
                     out_hbm.at[pl.ds((base + s) * lines_per_blk,
                                      lines_per_blk)], osems[p])

  def wait_wb(s, p):
    pltpu.make_async_copy(dst_v.at[p],
                          out_hbm.at[pl.ds((base + s) * lines_per_blk,
                                           lines_per_blk)], osems[p]).wait()

  @pl.when(n > 0)
  def _prologue():
    fire_read(0, 0)

  def pair_body(i0, carry):
    for b in range(2):
      s = i0 + b
      @pl.when(s < n)
      def _slot():
        @pl.when(s + 1 < n)
        def _():
          fire_read(s + 1, (b + 1) % 2)
        wait_read(s, b)
        @pl.when(s >= 2)
        def _():
          wait_wb(s, b)
        transpose_cols(b)
        fire_wb(s, b)
    return carry

  lax.fori_loop(0, nslots // 2, lambda i, c: pair_body(i * 2, c), 0,
                unroll=False)
  for b in range(2):
    @pl.when(n > b)
    def _drain():
      wait_wb(0, b)

  tail = V - nblk * BB
  if tail:
    @pl.when(wid == NW - 1)
    def _tail():
      pltpu.sync_copy(tail_hbm, dst_v.at[0, pl.ds(0, tail // 4), :])
      pltpu.sync_copy(dst_v.at[0, pl.ds(0, tail // 4), :],
                      out_hbm.at[pl.ds(nblk * lines_per_blk, tail // 4)])


def _emb_body(H, D, x_hbm, table_hbm, out_hbm, idx_v, rows_v, t_v,
              gsem0, gsem1, gsem2, gsem3, wsem0, wsem1):
  wid = lax.axis_index("s") * NC + lax.axis_index("c")
  pltpu.sync_copy(x_hbm.at[wid], idx_v)

  iota = lax.broadcasted_iota(jnp.int32, (L,), 0)
  idx_d = [iota + L * k for k in range(D // L)]
  gsems = (gsem0, gsem1, gsem2, gsem3)
  wsems = (wsem0, wsem1)
  nt = D // 8

  def fire_gather(h, g):
    pltpu.async_copy(table_hbm.at[idx_v.at[h]], rows_v.at[g], gsems[g])

  def wait_gather(h, g):
    pltpu.make_async_copy(table_hbm.at[idx_v.at[h]], rows_v.at[g],
                          gsems[g]).wait()

  def transpose_block(g, p):
    src = rows_v.at[g]
    dst = t_v.at[p]

    def _row(r, carry):
      rvec = jnp.full((L,), r, dtype=jnp.int32)
      for k in range(D // L):
        val = src[r, pl.ds(L * k, L)]
        plsc.store_scatter(dst, [idx_d[k], rvec], val)
      return carry

    lax.fori_loop(0, BB, _row, 0, unroll=8)

  def fire_wb(h, p):
    for ti in range(nt):
      pltpu.async_copy(t_v.at[p, pl.ds(ti * 8, 8), pl.ds(0, BB)],
                       out_hbm.at[h, ti, wid], wsems[p])

  def wait_wb(h, p):
    for ti in range(nt):
      pltpu.make_async_copy(t_v.at[p, pl.ds(ti * 8, 8), pl.ds(0, BB)],
                            out_hbm.at[h, ti, wid], wsems[p]).wait()

  for h in range(GDEPTH):
    fire_gather(h, h)

  def quad_body(h0, carry):
    for b in range(GDEPTH):
      h = h0 + b
      p = b % 2
      wait_gather(h, b)
      @pl.when(h >= 2)
      def _drain_prev_wb():
        wait_wb(h, p)
      transpose_block(b, p)
      @pl.when(h + GDEPTH < H)
      def _fire_next():
        fire_gather(h + GDEPTH, b)
      fire_wb(h, p)
    return carry

  lax.fori_loop(0, H // GDEPTH, lambda i, c: quad_body(i * GDEPTH, c), 0,
                unroll=False)

  for p in range(2):
    wait_wb(0, p)


def kernel(x, table):
  B, H = x.shape
  V, D = table.shape
  assert B == NW * BB and D % 8 == 0 and H % GDEPTH == 0
  nt = D // 8

  mesh = plsc.VectorSubcoreMesh(core_axis_name="c", subcore_axis_name="s")
  nblk = V // BB

  relocate = pl.kernel(
      functools.partial(_relocate_body, V, D),
      out_type=jax.ShapeDtypeStruct((nblk, D, BB), jnp.float32),
      mesh=mesh,
      scratch_types=[
          pltpu.VMEM((2, D, BB), jnp.float32),
          pltpu.SemaphoreType.DMA,
          pltpu.SemaphoreType.DMA,
          pltpu.SemaphoreType.DMA,
          pltpu.SemaphoreType.DMA,
      ],
      compiler_params=pltpu.CompilerParams(use_tc_tiling_on_sc=True,
                                           needs_layout_passes=False),
  )

  transpose = pl.kernel(
      functools.partial(_transpose_body, V, D),
      out_type=jax.ShapeDtypeStruct((V // 4, 4 * D), jnp.float32),
      mesh=mesh,
      scratch_types=[
          pltpu.VMEM((2, D, PITCH), jnp.float32),
          pltpu.VMEM((2, BB // 4, 4 * D), jnp.float32),
          pltpu.SemaphoreType.DMA,
          pltpu.SemaphoreType.DMA,
          pltpu.SemaphoreType.DMA,
          pltpu.SemaphoreType.DMA,
      ],
      compiler_params=pltpu.CompilerParams(use_tc_tiling_on_sc=False,
                                           needs_layout_passes=False),
  )

  xq = x.T.reshape(H, NW, BB).transpose(1, 0, 2)
  tail = V % BB
  t_tail = table[V - tail:].reshape(tail // 4, 4 * D)
  raw = relocate(table.T)
  t32 = transpose(raw, t_tail).reshape(V, D)

  grid_kernel = pl.kernel(
      functools.partial(_emb_body, H, D),
      out_type=jax.ShapeDtypeStruct((H, nt, NW, 8, BB), jnp.float32),
      mesh=mesh,
      scratch_types=[
          pltpu.VMEM((H, BB), jnp.int32),
          pltpu.VMEM((GDEPTH, BB, D), jnp.float32),
          pltpu.VMEM((2, D, PITCH), jnp.float32),
          pltpu.SemaphoreType.DMA,
          pltpu.SemaphoreType.DMA,
          pltpu.SemaphoreType.DMA,
          pltpu.SemaphoreType.DMA,
          pltpu.SemaphoreType.DMA,
          pltpu.SemaphoreType.DMA,
      ],
      compiler_params=pltpu.CompilerParams(use_tc_tiling_on_sc=False,
                                           needs_layout_passes=False),
  )
  out5 = grid_kernel(xq, t32)
  return out5.transpose(2, 4, 0, 1, 3).reshape(B, H, D)


# speedup vs baseline: 1.5868x; 1.1729x over previous
import functools

import jax
import jax.numpy as jnp
from jax import lax
from jax.experimental import pallas as pl
from jax.experimental.pallas import tpu as pltpu
from jax.experimental.pallas import tpu_sc as plsc

NC = 2
NS = 16
NW = NC * NS
L = 16

BB = 128
PITCH = BB + 1
GDEPTH = 4


def _split_work(wid, total):
  per_w_lo = total // NW
  n_extra = total - per_w_lo * NW
  base = wid * per_w_lo + jnp.minimum(wid, n_extra)
  n = per_w_lo + jnp.where(wid < n_extra, 1, 0)
  nslots = per_w_lo + (1 if n_extra else 0)
  nslots += nslots % 2
  return base, n, nslots


def _relocate_body(V, D, tt_hbm, out_hbm, buf_v, rsem0, rsem1,
                   osem0, osem1):
  wid = lax.axis_index("s") * NC + lax.axis_index("c")
  nblk = V // BB
  base, n, nslots = _split_work(wid, nblk)
  rsems = (rsem0, rsem1)
  osems = (osem0, osem1)

  def fire_read(s, p):
    pltpu.async_copy(tt_hbm.at[:, pl.ds((base + s) * BB, BB)],
                     buf_v.at[p], rsems[p])

  def wait_read(s, p):
    pltpu.make_async_copy(tt_hbm.at[:, pl.ds((base + s) * BB, BB)],
                          buf_v.at[p], rsems[p]).wait()

  def fire_wb(s, p):
    pltpu.async_copy(buf_v.at[p], out_hbm.at[base + s], osems[p])

  def wait_wb(s, p):
    pltpu.make_async_copy(buf_v.at[p], out_hbm.at[base + s],
                          osems[p]).wait()

  @pl.when(n > 0)
  def _prologue():
    fire_read(0, 0)

  def pair_body(i0, carry):
    for b in range(2):
      s = i0 + b
      @pl.when(s < n)
      def _slot():
        @pl.when(s + 1 < n)
        def _():
          fire_read(s + 1, (b + 1) % 2)
        wait_read(s, b)
        @pl.when(s >= 2)
        def _():
          wait_wb(s, b)
        fire_wb(s, b)
    return carry

  lax.fori_loop(0, nslots // 2, lambda i, c: pair_body(i * 2, c), 0,
                unroll=False)
  for b in range(2):
    @pl.when(n > b)
    def _drain():
      wait_wb(0, b)


def _transpose_body(V, D, raw_hbm, tail_hbm, out_hbm, src_v, dst_v,
                    rsem0, rsem1, osem0, osem1):
  wid = lax.axis_index("s") * NC + lax.axis_index("c")
  nblk = V // BB
  base, n, nslots = _split_work(wid, nblk)
  lines_per_blk = BB // 4

  iota = lax.broadcasted_iota(jnp.int32, (L,), 0)
  line_vec = [(iota // 4) + 4 * j for j in range(BB // L)]
  posbase = (iota % 4) * D
  rsems = (rsem0, rsem1)
  osems = (osem0, osem1)

  def fire_read(s, p):
    pltpu.async_copy(raw_hbm.at[base + s], src_v.at[p], rsems[p])

  def wait_read(s, p):
    pltpu.make_async_copy(raw_hbm.at[base + s], src_v.at[p],
                          rsems[p]).wait()

  def transpose_cols(p):
    src = src_v.at[p]
    dst = dst_v.at[p]

    def _row(d, carry):
      pos = posbase + jnp.full((L,), d, dtype=jnp.int32)
      for j in range(BB // L):
        val = src[d, pl.ds(L * j, L)]
        plsc.store_scatter(dst, [line_vec[j], pos], val)
      return carry

    lax.fori_loop(0, D, _row, 0, unroll=4)

  def fire_wb(s, p):
    pltpu.async_copy(dst_v.at[p, :, pl.ds(0, 4 * D)],
                     out_hbm.at[pl.ds((base + s) * lines_per_blk,
                                      lines_per_blk)], osems[p])

  def wait_wb(s, p):
    pltpu.make_async_copy(dst_v.at[p, :, pl.ds(0, 4 * D)],
                          out_hbm.at[pl.ds((base + s) * lines_per_blk,
                                           lines_per_blk)], osems[p]).wait()

  @pl.when(n > 0)
  def _prologue():
    fire_read(0, 0)

  def pair_body(i0, carry):
    for b in range(2):
      s = i0 + b
      @pl.when(s < n)
      def _slot():
        @pl.when(s + 1 < n)
        def _():
          fire_read(s + 1, (b + 1) % 2)
        wait_read(s, b)
        @pl.when(s >= 2)
        def _():
          wait_wb(s, b)
        transpose_cols(b)
        fire_wb(s, b)
    return carry

  lax.fori_loop(0, nslots // 2, lambda i, c: pair_body(i * 2, c), 0,
                unroll=False)
  for b in range(2):
    @pl.when(n > b)
    def _drain():
      wait_wb(0, b)

  tail = V - nblk * BB
  if tail:
    @pl.when(wid == NW - 1)
    def _tail():
      pltpu.sync_copy(tail_hbm,
                      dst_v.at[0, pl.ds(0, tail // 4), pl.ds(0, 4 * D)])
      pltpu.sync_copy(dst_v.at[0, pl.ds(0, tail // 4), pl.ds(0, 4 * D)],
                      out_hbm.at[pl.ds(nblk * lines_per_blk, tail // 4)])


def _emb_body(H, D, x_hbm, table_hbm, out_hbm, idx_v, rows_v, t_v,
              gsem0, gsem1, gsem2, gsem3, wsem0, wsem1):
  wid = lax.axis_index("s") * NC + lax.axis_index("c")
  pltpu.sync_copy(x_hbm.at[wid], idx_v)

  iota = lax.broadcasted_iota(jnp.int32, (L,), 0)
  idx_d = [iota + L * k for k in range(D // L)]
  gsems = (gsem0, gsem1, gsem2, gsem3)
  wsems = (wsem0, wsem1)
  nt = D // 8

  def fire_gather(h, g):
    pltpu.async_copy(table_hbm.at[idx_v.at[h]], rows_v.at[g], gsems[g])

  def wait_gather(h, g):
    pltpu.make_async_copy(table_hbm.at[idx_v.at[h]], rows_v.at[g],
                          gsems[g]).wait()

  def transpose_block(g, p):
    src = rows_v.at[g]
    dst = t_v.at[p]

    def _row(r, carry):
      rvec = jnp.full((L,), r, dtype=jnp.int32)
      for k in range(D // L):
        val = src[r, pl.ds(L * k, L)]
        plsc.store_scatter(dst, [idx_d[k], rvec], val)
      return carry

    lax.fori_loop(0, BB, _row, 0, unroll=8)

  def fire_wb(h, p):
    for ti in range(nt):
      pltpu.async_copy(t_v.at[p, pl.ds(ti * 8, 8), pl.ds(0, BB)],
                       out_hbm.at[h, ti, wid], wsems[p])

  def wait_wb(h, p):
    for ti in range(nt):
      pltpu.make_async_copy(t_v.at[p, pl.ds(ti * 8, 8), pl.ds(0, BB)],
                            out_hbm.at[h, ti, wid], wsems[p]).wait()

  for h in range(GDEPTH):
    fire_gather(h, h)

  def quad_body(h0, carry):
    for b in range(GDEPTH):
      h = h0 + b
      p = b % 2
      wait_gather(h, b)
      @pl.when(h >= 2)
      def _drain_prev_wb():
        wait_wb(h, p)
      transpose_block(b, p)
      @pl.when(h + GDEPTH < H)
      def _fire_next():
        fire_gather(h + GDEPTH, b)
      fire_wb(h, p)
    return carry

  lax.fori_loop(0, H // GDEPTH, lambda i, c: quad_body(i * GDEPTH, c), 0,
                unroll=False)

  for p in range(2):
    wait_wb(0, p)


def kernel(x, table):
  B, H = x.shape
  V, D = table.shape
  assert B == NW * BB and D % 8 == 0 and H % GDEPTH == 0
  nt = D // 8

  mesh = plsc.VectorSubcoreMesh(core_axis_name="c", subcore_axis_name="s")
  nblk = V // BB

  relocate = pl.kernel(
      functools.partial(_relocate_body, V, D),
      out_type=jax.ShapeDtypeStruct((nblk, D, BB), jnp.float32),
      mesh=mesh,
      scratch_types=[
          pltpu.VMEM((2, D, BB), jnp.float32),
          pltpu.SemaphoreType.DMA,
          pltpu.SemaphoreType.DMA,
          pltpu.SemaphoreType.DMA,
          pltpu.SemaphoreType.DMA,
      ],
      compiler_params=pltpu.CompilerParams(use_tc_tiling_on_sc=True,
                                           needs_layout_passes=False),
  )

  transpose = pl.kernel(
      functools.partial(_transpose_body, V, D),
      out_type=jax.ShapeDtypeStruct((V // 4, 4 * D), jnp.float32),
      mesh=mesh,
      scratch_types=[
          pltpu.VMEM((2, D, BB), jnp.float32),
          pltpu.VMEM((2, BB // 4, 4 * D + 4), jnp.float32),
          pltpu.SemaphoreType.DMA,
          pltpu.SemaphoreType.DMA,
          pltpu.SemaphoreType.DMA,
          pltpu.SemaphoreType.DMA,
      ],
      compiler_params=pltpu.CompilerParams(use_tc_tiling_on_sc=False,
                                           needs_layout_passes=False),
  )

  xq = x.T.reshape(H, NW, BB).transpose(1, 0, 2)
  tail = V % BB
  t_tail = table[V - tail:].reshape(tail // 4, 4 * D)
  raw = relocate(table.T)
  t32 = transpose(raw, t_tail).reshape(V, D)

  grid_kernel = pl.kernel(
      functools.partial(_emb_body, H, D),
      out_type=jax.ShapeDtypeStruct((H, nt, NW, 8, BB), jnp.float32),
      mesh=mesh,
      scratch_types=[
          pltpu.VMEM((H, BB), jnp.int32),
          pltpu.VMEM((GDEPTH, BB, D), jnp.float32),
          pltpu.VMEM((2, D, PITCH), jnp.float32),
          pltpu.SemaphoreType.DMA,
          pltpu.SemaphoreType.DMA,
          pltpu.SemaphoreType.DMA,
          pltpu.SemaphoreType.DMA,
          pltpu.SemaphoreType.DMA,
          pltpu.SemaphoreType.DMA,
      ],
      compiler_params=pltpu.CompilerParams(use_tc_tiling_on_sc=False,
                                           needs_layout_passes=False),
  )
  out5 = grid_kernel(xq, t32)
  return out5.transpose(2, 4, 0, 1, 3).reshape(B, H, D)
